# baseline (speedup 1.0000x reference)
import functools

import jax
import jax.numpy as jnp
from jax import lax
from jax.experimental import pallas as pl
from jax.experimental.pallas import tpu as pltpu
from jax.experimental.pallas import tpu_sc as plsc

BATCH = 16384
VOCAB = 1000000
HIDDEN = 64
NUM_SAMPLE = 5

NUM_WORKERS = 32
BW = BATCH // NUM_WORKERS
CB = 128
PC = CB * NUM_SAMPLE
NCHUNK = BW // CB
GN = 128
SB = 16


def _negsamp_body(inputs_hbm, idx_hbm, table_hbm, out_hbm,
                  idx_v, rows_v, in_v, out_s, mat, sem):
    cid = lax.axis_index("c")
    sid = lax.axis_index("s")
    wid = sid * 2 + cid
    pair0 = wid * (BW * NUM_SAMPLE)
    b0 = wid * BW

    pltpu.sync_copy(idx_hbm.at[pl.ds(pair0, BW * NUM_SAMPLE)], idx_v)

    for c in range(NCHUNK):
        handles = []
        for j in range(PC // GN):
            handles.append(pltpu.async_copy(
                table_hbm.at[idx_v.at[pl.ds(c * PC + j * GN, GN)]],
                rows_v.at[pl.ds(j * GN, GN)],
                sem,
            ))
        pltpu.sync_copy(inputs_hbm.at[pl.ds(b0 + c * CB, CB)], in_v)
        for h in handles:
            h.wait()

        tbase = lax.iota(jnp.int32, 16) * 16

        def sb_body(g, _):
            for bb in range(SB):
                b = g * SB + bb
                xs = [in_v[b, pl.ds(k * 16, 16)] for k in range(HIDDEN // 16)]
                for s in range(NUM_SAMPLE):
                    p = b * NUM_SAMPLE + s
                    acc = rows_v[p, pl.ds(0, 16)] * xs[0]
                    for k in range(1, HIDDEN // 16):
                        acc = acc + rows_v[p, pl.ds(k * 16, 16)] * xs[k]
                    mat[pl.ds((bb * NUM_SAMPLE + s) * 16, 16)] = acc
            for j in range(SB * NUM_SAMPLE // 16):
                tj = tbase + (256 * j)
                dv = plsc.load_gather(mat, [tj])
                for l in range(1, 16):
                    dv = dv + plsc.load_gather(mat, [tj + l])
                sig = 1.0 / (1.0 + jnp.exp(-dv))
                out_s[pl.ds(g * (SB * NUM_SAMPLE) + j * 16, 16)] = sig
            return 0

        lax.fori_loop(0, CB // SB, sb_body, 0)

        pltpu.sync_copy(out_s, out_hbm.at[pl.ds(pair0 + c * PC, PC)])


@functools.partial(jax.jit, donate_argnums=())
def _negsamp(inputs, idx_flat, table):
    mesh = plsc.VectorSubcoreMesh(core_axis_name="c", subcore_axis_name="s")
    f = pl.kernel(
        _negsamp_body,
        mesh=mesh,
        out_type=jax.ShapeDtypeStruct((BATCH * NUM_SAMPLE,), jnp.float32),
        scratch_types=[
            pltpu.VMEM((BW * NUM_SAMPLE,), jnp.int32),
            pltpu.VMEM((PC, HIDDEN), jnp.float32),
            pltpu.VMEM((CB, HIDDEN), jnp.float32),
            pltpu.VMEM((PC,), jnp.float32),
            pltpu.VMEM((SB * NUM_SAMPLE * 16,), jnp.float32),
            pltpu.SemaphoreType.DMA,
        ],
        compiler_params=pltpu.CompilerParams(
            needs_layout_passes=False,
            use_tc_tiling_on_sc=False,
        ),
    )
    return f(inputs, idx_flat, table)


def kernel(inputs, idxs, out_embedding):
    idx_flat = idxs.reshape(-1).astype(jnp.int32)
    out = _negsamp(inputs, idx_flat, out_embedding)
    return out.reshape(BATCH, NUM_SAMPLE)


# speedup vs baseline: 2.4626x; 2.4626x over previous
import functools

import jax
import jax.numpy as jnp
from jax import lax
from jax.experimental import pallas as pl
from jax.experimental.pallas import tpu as pltpu
from jax.experimental.pallas import tpu_sc as plsc

BATCH = 16384
VOCAB = 1000000
HIDDEN = 64
NUM_SAMPLE = 5

NPAIR = BATCH * NUM_SAMPLE
NTILE = 16
PT = NPAIR // NTILE
BT = PT // NUM_SAMPLE
NJ = PT // 128
HC = HIDDEN // 2


def _planes_body(inputsT_hbm, idx_hbm, tableT_hbm, part_hbm,
                 idx_v, biv, val_v, acc_v, inp_v, sp,
                 sem_p, sem_g, sem_i):
    c = lax.axis_index("c")
    s = lax.axis_index("s")
    h0 = c * HC
    b0 = s * BT

    pltpu.sync_copy(idx_hbm.at[s], idx_v)

    lane = lax.iota(jnp.int32, 16)

    def init_body(j, _):
        for l in range(8):
            sl = pl.ds(l * 16, 16)
            base = j * 128 + l * 16
            biv[j, sl] = (base + lane) // NUM_SAMPLE
            acc_v[j, sl] = jnp.zeros((16,), jnp.float32)
        return 0

    lax.fori_loop(0, NJ, init_body, 0)

    @pl.when(s == 0)
    def _():
        pltpu.async_copy(tableT_hbm.at[h0], sp, sem_p)

    def plane_body(k, _):
        h = h0 + k

        @pl.when(s == 0)
        def _():
            pltpu.make_async_copy(tableT_hbm.at[h0], sp, sem_p).wait()

        plsc.subcore_barrier()

        inp_cp = pltpu.async_copy(
            inputsT_hbm.at[h, pl.ds(b0, BT)], inp_v, sem_i)
        gathers = [
            pltpu.async_copy(sp.at[idx_v.at[j]], val_v.at[j], sem_g)
            for j in range(NJ)
        ]
        inp_cp.wait()
        for g in gathers:
            g.wait()

        plsc.subcore_barrier()

        @pl.when((s == 0) & (k < HC - 1))
        def _():
            pltpu.async_copy(tableT_hbm.at[h + 1], sp, sem_p)

        def comp(j, _):
            for l in range(8):
                sl = pl.ds(l * 16, 16)
                x = plsc.load_gather(inp_v, [biv[j, sl]])
                acc_v[j, sl] = acc_v[j, sl] + val_v[j, sl] * x
            return 0

        lax.fori_loop(0, NJ, comp, 0)
        return 0

    lax.fori_loop(0, HC, plane_body, 0)

    pltpu.sync_copy(acc_v, part_hbm.at[c, s])


@jax.jit
def _planes(inputsT, idx3, tableT):
    mesh = plsc.VectorSubcoreMesh(core_axis_name="c", subcore_axis_name="s")
    f = pl.kernel(
        _planes_body,
        mesh=mesh,
        out_type=jax.ShapeDtypeStruct((2, NTILE, NJ, 128), jnp.float32),
        scratch_types=[
            pltpu.VMEM((NJ, 128), jnp.int32),
            pltpu.VMEM((NJ, 128), jnp.int32),
            pltpu.VMEM((NJ, 128), jnp.float32),
            pltpu.VMEM((NJ, 128), jnp.float32),
            pltpu.VMEM((BT,), jnp.float32),
            pltpu.VMEM_SHARED((VOCAB,), jnp.float32),
            pltpu.SemaphoreType.DMA,
            pltpu.SemaphoreType.DMA,
            pltpu.SemaphoreType.DMA,
        ],
        compiler_params=pltpu.CompilerParams(needs_layout_passes=False),
    )
    return f(inputsT, idx3, tableT)


def _combine_body(p_ref, o_ref):
    z = p_ref[0] + p_ref[1]
    o_ref[...] = 1.0 / (1.0 + jnp.exp(-z))


@jax.jit
def _combine(part):
    return pl.pallas_call(
        _combine_body,
        out_shape=jax.ShapeDtypeStruct((NPAIR // 128, 128), jnp.float32),
    )(part)


def kernel(inputs, idxs, out_embedding):
    tableT = out_embedding.T
    inputsT = inputs.T
    idx3 = idxs.reshape(-1).astype(jnp.int32).reshape(NTILE, NJ, 128)
    part = _planes(inputsT, idx3, tableT)
    out = _combine(part.reshape(2, NPAIR // 128, 128))
    return out.reshape(BATCH, NUM_SAMPLE)
